# baseline (speedup 1.0000x reference)
import functools
import math

import jax
import jax.numpy as jnp
from jax import lax
from jax.experimental import pallas as pl
from jax.experimental.pallas import tpu as pltpu
from jax.experimental.pallas import tpu_sc as plsc

_NC = 2
_NS = 16
_NW = _NC * _NS

_SUB = 128
_SUBS_PER_CHUNK = 4
_CHUNK = _SUB * _SUBS_PER_CHUNK


def _build_gather(n_rows: int, dim: int, scale: float):
    assert n_rows % (_NW * _CHUNK) == 0
    rows_per_w = n_rows // _NW
    n_chunks = rows_per_w // _CHUNK
    idx_rows_per_w = rows_per_w // _SUB

    mesh = plsc.VectorSubcoreMesh(core_axis_name="c", subcore_axis_name="s")

    @functools.partial(
        pl.kernel,
        mesh=mesh,
        compiler_params=pltpu.CompilerParams(use_tc_tiling_on_sc=False),
        out_type=jax.ShapeDtypeStruct((n_rows, dim), jnp.float32),
        scratch_types=[
            pltpu.VMEM((_SUBS_PER_CHUNK, _SUB), jnp.int32),
            pltpu.VMEM((_CHUNK, dim), jnp.float32),
            pltpu.SemaphoreType.DMA,
        ],
    )
    def k(table_hbm, idx_hbm, out_hbm, idx_v, rows_v, sem):
        wid = lax.axis_index("s") * _NC + lax.axis_index("c")
        idx_row_base = wid * idx_rows_per_w
        out_row_base = wid * rows_per_w

        def mul_body(i, carry):
            for j in range(dim // 16):
                sl = pl.ds(j * 16, 16)
                rows_v[i, sl] = rows_v[i, sl] * scale
            return carry

        def chunk_body(g, carry):
            pltpu.sync_copy(
                idx_hbm.at[pl.ds(idx_row_base + g * _SUBS_PER_CHUNK,
                                 _SUBS_PER_CHUNK)],
                idx_v)
            cps = [
                pltpu.async_copy(table_hbm.at[idx_v.at[j]],
                                 rows_v.at[pl.ds(j * _SUB, _SUB)], sem)
                for j in range(_SUBS_PER_CHUNK)
            ]
            for cp in cps:
                cp.wait()
            lax.fori_loop(0, _CHUNK, mul_body, 0)
            pltpu.sync_copy(
                rows_v,
                out_hbm.at[pl.ds(out_row_base + g * _CHUNK, _CHUNK)])
            return carry

        lax.fori_loop(0, n_chunks, chunk_body, 0)

    return k


def kernel(token_ids_batch, table):
    batch, seq = token_ids_batch.shape
    vocab, dim = table.shape
    n_rows = batch * seq
    idx2d = token_ids_batch.reshape(n_rows // _SUB, _SUB).astype(jnp.int32)
    scale = math.sqrt(dim)
    out = _build_gather(n_rows, dim, scale)(table, idx2d)
    return out.reshape(batch, seq, dim)


# speedup vs baseline: 1.1322x; 1.1322x over previous
import functools
import math

import jax
import jax.numpy as jnp
from jax import lax
from jax.experimental import pallas as pl
from jax.experimental.pallas import tpu as pltpu
from jax.experimental.pallas import tpu_sc as plsc

_NC = 2
_NS = 16
_NW = _NC * _NS

_SUB = 128
_SUBS_PER_CHUNK = 4
_CHUNK = _SUB * _SUBS_PER_CHUNK


def _build_gather(n_rows: int, dim: int, scale: float):
    assert n_rows % (_NW * 2 * _CHUNK) == 0
    rows_per_w = n_rows // _NW
    n_chunks = rows_per_w // _CHUNK
    idx_rows_per_w = rows_per_w // _SUB

    mesh = plsc.VectorSubcoreMesh(core_axis_name="c", subcore_axis_name="s")

    @functools.partial(
        pl.kernel,
        mesh=mesh,
        compiler_params=pltpu.CompilerParams(use_tc_tiling_on_sc=False),
        out_type=jax.ShapeDtypeStruct((n_rows, dim), jnp.float32),
        scratch_types=[
            pltpu.VMEM((idx_rows_per_w, _SUB), jnp.int32),
            pltpu.VMEM((_CHUNK, dim), jnp.float32),
            pltpu.VMEM((_CHUNK, dim), jnp.float32),
            pltpu.SemaphoreType.DMA,
            pltpu.SemaphoreType.DMA,
        ],
    )
    def k(table_hbm, idx_hbm, out_hbm, idx_v, rows0, rows1, sem0, sem1):
        wid = lax.axis_index("s") * _NC + lax.axis_index("c")
        out_row_base = wid * rows_per_w

        pltpu.sync_copy(idx_hbm.at[pl.ds(wid * idx_rows_per_w,
                                         idx_rows_per_w)], idx_v)

        def start_gather(c, rows, sem):
            for j in range(_SUBS_PER_CHUNK):
                pltpu.async_copy(
                    table_hbm.at[idx_v.at[c * _SUBS_PER_CHUNK + j]],
                    rows.at[pl.ds(j * _SUB, _SUB)], sem)

        def mul_body(rows):
            def body(i, carry):
                for j in range(dim // 16):
                    sl = pl.ds(j * 16, 16)
                    rows[i, sl] = rows[i, sl] * scale
                return carry
            lax.fori_loop(0, _CHUNK, body, 0, unroll=8)

        def process(g, rows, sem):
            pltpu.make_async_copy(table_hbm.at[pl.ds(0, _CHUNK)], rows,
                                  sem).wait()
            mul_body(rows)
            pltpu.sync_copy(rows,
                            out_hbm.at[pl.ds(out_row_base + g * _CHUNK,
                                             _CHUNK)])
            nxt = g + 2

            @pl.when(nxt < n_chunks)
            def _():
                start_gather(nxt, rows, sem)

        start_gather(0, rows0, sem0)
        start_gather(1, rows1, sem1)

        def loop_body(g2, carry):
            process(2 * g2, rows0, sem0)
            process(2 * g2 + 1, rows1, sem1)
            return carry

        lax.fori_loop(0, n_chunks // 2, loop_body, 0)

    return k


def kernel(token_ids_batch, table):
    batch, seq = token_ids_batch.shape
    vocab, dim = table.shape
    n_rows = batch * seq
    idx2d = token_ids_batch.reshape(n_rows // _SUB, _SUB).astype(jnp.int32)
    scale = math.sqrt(dim)
    out = _build_gather(n_rows, dim, scale)(table, idx2d)
    return out.reshape(batch, seq, dim)
